# baseline (speedup 1.0000x reference)
import functools

import jax
import jax.numpy as jnp
from jax import lax
from jax.experimental import pallas as pl
from jax.experimental.pallas import tpu as pltpu
from jax.experimental.pallas import tpu_sc as plsc

DIM = 32

_info = plsc.get_sparse_core_info()
_NC, _NS = _info.num_cores, _info.num_subcores
NW = _NC * _NS


def _make_gather_tiled(n_seq: int, n_pos: int):
    chunk = n_seq // NW
    n_itl = chunk // 128
    n_ct = DIM // 8
    face = DIM * n_seq
    mesh = plsc.VectorSubcoreMesh(core_axis_name="c", subcore_axis_name="s")

    @functools.partial(
        pl.kernel,
        mesh=mesh,
        out_type=jax.ShapeDtypeStruct((n_pos * DIM * n_seq,), jnp.float32),
        scratch_types=[
            pltpu.VMEM((2, chunk), jnp.int32),
            pltpu.VMEM((2, chunk, DIM), jnp.float32),
            pltpu.VMEM((2, n_ct * n_itl * 8 * 128), jnp.float32),
        ]
        + [pltpu.SemaphoreType.DMA] * 6,
        compiler_params=pltpu.CompilerParams(
            use_tc_tiling_on_sc=False, needs_layout_passes=False
        ),
    )
    def gather_kernel(idx_hbm, table_hbm, out_hbm, idx_v, rows_v, t_v, *sems):
        isem = sems[0:2]
        gsem = sems[2:4]
        osem = sems[4:6]
        wid = lax.axis_index("s") * _NC + lax.axis_index("c")
        seq0 = wid * chunk
        iota = lax.iota(jnp.int32, 16)

        def idx_start(j, b):
            pltpu.async_copy(
                idx_hbm.at[pl.ds(j * n_seq + seq0, chunk)], idx_v.at[b], isem[b]
            )

        def idx_wait(b):
            pltpu.make_async_copy(
                idx_hbm.at[pl.ds(0, chunk)], idx_v.at[b], isem[b]
            ).wait()

        def gather_start(b):
            pltpu.async_copy(table_hbm.at[idx_v.at[b]], rows_v.at[b], gsem[b])

        def gather_wait(b):
            pltpu.make_async_copy(
                table_hbm.at[idx_v.at[b]], rows_v.at[b], gsem[b]
            ).wait()

        def wb_start(j, b):
            for ct in range(n_ct):
                pltpu.async_copy(
                    t_v.at[b].at[pl.ds(ct * n_itl * 1024, n_itl * 1024)],
                    out_hbm.at[
                        pl.ds(
                            j * face + (ct * (n_seq // 128) + wid * n_itl) * 1024,
                            n_itl * 1024,
                        )
                    ],
                    osem[b],
                )

        def wb_wait(b):
            for ct in range(n_ct):
                pltpu.make_async_copy(
                    t_v.at[b].at[pl.ds(0, n_itl * 1024)],
                    out_hbm.at[pl.ds(0, n_itl * 1024)],
                    osem[b],
                ).wait()

        def transpose(b):
            rows = rows_v.at[b]
            tb = t_v.at[b]

            def q_body(q, carry):
                itl = q % n_itl
                ct = q // n_itl
                rvecs = [iota + (itl * 128 + v * 16) for v in range(8)]
                cvecs = [
                    jnp.full((16,), 0, jnp.int32) + (8 * ct + cr)
                    for cr in range(8)
                ]
                base = q * 1024
                for cr in range(8):
                    for v in range(8):
                        val = plsc.load_gather(rows, [rvecs[v], cvecs[cr]])
                        tb[pl.ds(base + cr * 128 + v * 16, 16)] = val
                return carry

            lax.fori_loop(0, n_ct * n_itl, q_body, 0)

        def step(j, b, first, start_next_gather, issue_idx):
            if start_next_gather:
                idx_wait(1 - b)
                gather_start(1 - b)
            gather_wait(b)
            if not first:
                wb_wait(b)
            transpose(b)
            wb_start(j, b)
            if issue_idx:
                idx_start(j + 2, b)

        idx_start(0, 0)
        idx_start(1, 1)
        idx_wait(0)
        gather_start(0)

        step(0, 0, first=True, start_next_gather=True, issue_idx=True)
        step(1, 1, first=True, start_next_gather=True, issue_idx=True)

        def outer(jj, carry):
            j0 = 2 * jj
            for b in range(2):
                j = j0 + b
                idx_wait(1 - b)
                gather_start(1 - b)
                gather_wait(b)
                wb_wait(b)
                transpose(b)
                wb_start(j, b)
                idx_start(j + 2, b)
            return carry

        lax.fori_loop(1, n_pos // 2 - 1, outer, 0)

        step(n_pos - 2, 0, first=False, start_next_gather=True, issue_idx=False)
        step(n_pos - 1, 1, first=False, start_next_gather=False, issue_idx=False)

        wb_wait(0)
        wb_wait(1)

    return gather_kernel


def kernel(token_ids, embedding_table):
    n_seq, n_pos = token_ids.shape
    B = n_seq * n_pos
    idx = token_ids.T.reshape(B).astype(jnp.int32)
    flat = _make_gather_tiled(n_seq, n_pos)(idx, embedding_table)
    out5 = flat.reshape(n_pos, DIM // 8, n_seq // 128, 8, 128)
    return out5.transpose(2, 4, 0, 1, 3).reshape(n_seq, n_pos, DIM)


# speedup vs baseline: 1.1821x; 1.1821x over previous
import functools

import jax
import jax.numpy as jnp
from jax import lax
from jax.experimental import pallas as pl
from jax.experimental.pallas import tpu as pltpu
from jax.experimental.pallas import tpu_sc as plsc

DIM = 32

_info = plsc.get_sparse_core_info()
_NC, _NS = _info.num_cores, _info.num_subcores
NW = _NC * _NS


def _make_gather_tiled(n_seq: int, n_pos: int):
    chunk = n_seq // NW
    n_itl = chunk // 128
    n_ct = DIM // 8
    face = DIM * n_seq
    mesh = plsc.VectorSubcoreMesh(core_axis_name="c", subcore_axis_name="s")

    @functools.partial(
        pl.kernel,
        mesh=mesh,
        out_type=jax.ShapeDtypeStruct((n_pos * DIM * n_seq,), jnp.float32),
        scratch_types=[
            pltpu.VMEM((2, chunk), jnp.int32),
            pltpu.VMEM((2, chunk, DIM), jnp.float32),
            pltpu.VMEM((2, n_ct * n_itl * 8 * 128), jnp.float32),
        ]
        + [pltpu.SemaphoreType.DMA] * 6,
        compiler_params=pltpu.CompilerParams(
            use_tc_tiling_on_sc=False, needs_layout_passes=False
        ),
    )
    def gather_kernel(idx_hbm, table_hbm, out_hbm, idx_v, rows_v, t_v, *sems):
        isem = sems[0:2]
        gsem = sems[2:4]
        osem = sems[4:6]
        wid = lax.axis_index("s") * _NC + lax.axis_index("c")
        seq0 = wid * chunk
        iota = lax.iota(jnp.int32, 16)

        def idx_start(j, b):
            pltpu.async_copy(
                idx_hbm.at[pl.ds(j * n_seq + seq0, chunk)], idx_v.at[b], isem[b]
            )

        def idx_wait(b):
            pltpu.make_async_copy(
                idx_hbm.at[pl.ds(0, chunk)], idx_v.at[b], isem[b]
            ).wait()

        def gather_start(b):
            pltpu.async_copy(table_hbm.at[idx_v.at[b]], rows_v.at[b], gsem[b])

        def gather_wait(b):
            pltpu.make_async_copy(
                table_hbm.at[idx_v.at[b]], rows_v.at[b], gsem[b]
            ).wait()

        def wb_start(j, b):
            for ct in range(n_ct):
                pltpu.async_copy(
                    t_v.at[b].at[pl.ds(ct * n_itl * 1024, n_itl * 1024)],
                    out_hbm.at[
                        pl.ds(
                            j * face + (ct * (n_seq // 128) + wid * n_itl) * 1024,
                            n_itl * 1024,
                        )
                    ],
                    osem[b],
                )

        def wb_wait(b):
            for ct in range(n_ct):
                pltpu.make_async_copy(
                    t_v.at[b].at[pl.ds(0, n_itl * 1024)],
                    out_hbm.at[pl.ds(0, n_itl * 1024)],
                    osem[b],
                ).wait()

        def transpose(b):
            rows = rows_v.at[b]
            tb = t_v.at[b]

            def q_body(q, carry):
                itl = q % n_itl
                ct = q // n_itl
                rvecs = [iota + (itl * 128 + v * 16) for v in range(8)]
                cvecs = [
                    jnp.full((16,), 0, jnp.int32) + (8 * ct + cr)
                    for cr in range(8)
                ]
                base = q * 1024
                for cr in range(8):
                    vals = [
                        plsc.load_gather(rows, [rvecs[v], cvecs[cr]])
                        for v in range(8)
                    ]
                    for v in range(8):
                        tb[pl.ds(base + cr * 128 + v * 16, 16)] = vals[v]
                return carry

            lax.fori_loop(0, n_ct * n_itl, q_body, 0)

        def step(j, b, first, start_next_gather, issue_idx):
            if start_next_gather:
                idx_wait(1 - b)
                gather_start(1 - b)
            gather_wait(b)
            if not first:
                wb_wait(b)
            transpose(b)
            wb_start(j, b)
            if issue_idx:
                idx_start(j + 2, b)

        idx_start(0, 0)
        idx_start(1, 1)
        idx_wait(0)
        gather_start(0)

        step(0, 0, first=True, start_next_gather=True, issue_idx=True)
        step(1, 1, first=True, start_next_gather=True, issue_idx=True)

        def outer(jj, carry):
            j0 = 2 * jj
            for b in range(2):
                j = j0 + b
                idx_wait(1 - b)
                gather_start(1 - b)
                gather_wait(b)
                wb_wait(b)
                transpose(b)
                wb_start(j, b)
                idx_start(j + 2, b)
            return carry

        lax.fori_loop(1, n_pos // 2 - 1, outer, 0)

        step(n_pos - 2, 0, first=False, start_next_gather=True, issue_idx=False)
        step(n_pos - 1, 1, first=False, start_next_gather=False, issue_idx=False)

        wb_wait(0)
        wb_wait(1)

    return gather_kernel


def kernel(token_ids, embedding_table):
    n_seq, n_pos = token_ids.shape
    B = n_seq * n_pos
    idx = token_ids.T.reshape(B).astype(jnp.int32)
    flat = _make_gather_tiled(n_seq, n_pos)(idx, embedding_table)
    out5 = flat.reshape(n_pos, DIM // 8, n_seq // 128, 8, 128)
    return out5.transpose(2, 4, 0, 1, 3).reshape(n_seq, n_pos, DIM)


# speedup vs baseline: 1.2456x; 1.0537x over previous
import functools

import jax
import jax.numpy as jnp
from jax import lax
from jax.experimental import pallas as pl
from jax.experimental.pallas import tpu as pltpu
from jax.experimental.pallas import tpu_sc as plsc

DIM = 32

_info = plsc.get_sparse_core_info()
_NC, _NS = _info.num_cores, _info.num_subcores
NW = _NC * _NS


def _make_gather_tiled(n_seq: int, n_pos: int):
    chunk = n_seq // NW
    n_itl = chunk // 128
    n_ct = DIM // 8
    face = DIM * n_seq
    mesh = plsc.VectorSubcoreMesh(core_axis_name="c", subcore_axis_name="s")

    @functools.partial(
        pl.kernel,
        mesh=mesh,
        out_type=jax.ShapeDtypeStruct((n_pos * DIM * n_seq,), jnp.float32),
        scratch_types=[
            pltpu.VMEM((2, chunk), jnp.int32),
            pltpu.VMEM((2, chunk, DIM), jnp.float32),
            pltpu.VMEM((2, n_ct * n_itl * 8 * 128), jnp.float32),
        ]
        + [pltpu.SemaphoreType.DMA] * 6,
        compiler_params=pltpu.CompilerParams(
            use_tc_tiling_on_sc=False, needs_layout_passes=False
        ),
    )
    def gather_kernel(idx_hbm, table_hbm, out_hbm, idx_v, rows_v, t_v, *sems):
        isem = sems[0:2]
        gsem = sems[2:4]
        osem = sems[4:6]
        wid = lax.axis_index("s") * _NC + lax.axis_index("c")
        seq0 = wid * chunk
        iota = lax.iota(jnp.int32, 16)

        def idx_start(j, b):
            pltpu.async_copy(
                idx_hbm.at[pl.ds(j * n_seq + seq0, chunk)], idx_v.at[b], isem[b]
            )

        def idx_wait(b):
            pltpu.make_async_copy(
                idx_hbm.at[pl.ds(0, chunk)], idx_v.at[b], isem[b]
            ).wait()

        def gather_start(b):
            pltpu.async_copy(table_hbm.at[idx_v.at[b]], rows_v.at[b], gsem[b])

        def gather_wait(b):
            pltpu.make_async_copy(
                table_hbm.at[idx_v.at[b]], rows_v.at[b], gsem[b]
            ).wait()

        def wb_start(j, b):
            for ct in range(n_ct):
                pltpu.async_copy(
                    t_v.at[b].at[pl.ds(ct * n_itl * 1024, n_itl * 1024)],
                    out_hbm.at[
                        pl.ds(
                            j * face + (ct * (n_seq // 128) + wid * n_itl) * 1024,
                            n_itl * 1024,
                        )
                    ],
                    osem[b],
                )

        def wb_wait(b):
            for ct in range(n_ct):
                pltpu.make_async_copy(
                    t_v.at[b].at[pl.ds(0, n_itl * 1024)],
                    out_hbm.at[pl.ds(0, n_itl * 1024)],
                    osem[b],
                ).wait()

        def transpose(b):
            rows = rows_v.at[b]
            tb = t_v.at[b]

            @plsc.parallel_loop(0, n_ct * n_itl, 1, unroll=2)
            def q_body(q):
                itl = q % n_itl
                ct = q // n_itl
                rvecs = [iota + (itl * 128 + v * 16) for v in range(8)]
                cvecs = [
                    jnp.full((16,), 0, jnp.int32) + (8 * ct + cr)
                    for cr in range(8)
                ]
                base = q * 1024
                for cr in range(8):
                    vals = [
                        plsc.load_gather(rows, [rvecs[v], cvecs[cr]])
                        for v in range(8)
                    ]
                    for v in range(8):
                        tb[pl.ds(base + cr * 128 + v * 16, 16)] = vals[v]

        def step(j, b, first, start_next_gather, issue_idx):
            if start_next_gather:
                idx_wait(1 - b)
                gather_start(1 - b)
            gather_wait(b)
            if not first:
                wb_wait(b)
            transpose(b)
            wb_start(j, b)
            if issue_idx:
                idx_start(j + 2, b)

        idx_start(0, 0)
        idx_start(1, 1)
        idx_wait(0)
        gather_start(0)

        step(0, 0, first=True, start_next_gather=True, issue_idx=True)
        step(1, 1, first=True, start_next_gather=True, issue_idx=True)

        def outer(jj, carry):
            j0 = 2 * jj
            for b in range(2):
                j = j0 + b
                idx_wait(1 - b)
                gather_start(1 - b)
                gather_wait(b)
                wb_wait(b)
                transpose(b)
                wb_start(j, b)
                idx_start(j + 2, b)
            return carry

        lax.fori_loop(1, n_pos // 2 - 1, outer, 0)

        step(n_pos - 2, 0, first=False, start_next_gather=True, issue_idx=False)
        step(n_pos - 1, 1, first=False, start_next_gather=False, issue_idx=False)

        wb_wait(0)
        wb_wait(1)

    return gather_kernel


def kernel(token_ids, embedding_table):
    n_seq, n_pos = token_ids.shape
    B = n_seq * n_pos
    idx = token_ids.T.reshape(B).astype(jnp.int32)
    flat = _make_gather_tiled(n_seq, n_pos)(idx, embedding_table)
    out5 = flat.reshape(n_pos, DIM // 8, n_seq // 128, 8, 128)
    return out5.transpose(2, 4, 0, 1, 3).reshape(n_seq, n_pos, DIM)


# speedup vs baseline: 1.3877x; 1.1140x over previous
import functools

import jax
import jax.numpy as jnp
from jax import lax
from jax.experimental import pallas as pl
from jax.experimental.pallas import tpu as pltpu
from jax.experimental.pallas import tpu_sc as plsc

DIM = 32

_info = plsc.get_sparse_core_info()
_NC, _NS = _info.num_cores, _info.num_subcores
NW = _NC * _NS


def _make_gather(B: int, chunk: int, nbuf: int):
    b_per_w = B // NW
    n_chunks = b_per_w // chunk
    n_outer = n_chunks // nbuf
    assert b_per_w % chunk == 0 and n_chunks % nbuf == 0 and n_outer >= 3
    mesh = plsc.VectorSubcoreMesh(core_axis_name="c", subcore_axis_name="s")

    @functools.partial(
        pl.kernel,
        mesh=mesh,
        out_type=jax.ShapeDtypeStruct((B, DIM), jnp.float32),
        scratch_types=[
            pltpu.VMEM((nbuf, chunk), jnp.int32),
            pltpu.VMEM((nbuf, chunk, DIM), jnp.float32),
        ]
        + [pltpu.SemaphoreType.DMA] * (3 * nbuf),
        compiler_params=pltpu.CompilerParams(use_tc_tiling_on_sc=False),
    )
    def gather_kernel(idx_hbm, table_hbm, out_hbm, idx_v, rows_v, *sems):
        isem = sems[0:nbuf]
        gsem = sems[nbuf : 2 * nbuf]
        osem = sems[2 * nbuf : 3 * nbuf]
        wid = lax.axis_index("s") * _NC + lax.axis_index("c")
        base = wid * b_per_w

        def idx_start(c, b):
            pltpu.async_copy(
                idx_hbm.at[pl.ds(base + c * chunk, chunk)], idx_v.at[b], isem[b]
            )

        def idx_wait(b):
            pltpu.make_async_copy(
                idx_hbm.at[pl.ds(0, chunk)], idx_v.at[b], isem[b]
            ).wait()

        def gather_start(b):
            pltpu.async_copy(table_hbm.at[idx_v.at[b]], rows_v.at[b], gsem[b])

        def gather_wait(b):
            pltpu.make_async_copy(
                table_hbm.at[idx_v.at[b]], rows_v.at[b], gsem[b]
            ).wait()

        def out_start(c, b):
            pltpu.async_copy(
                rows_v.at[b], out_hbm.at[pl.ds(base + c * chunk, chunk)], osem[b]
            )

        def out_wait(b):
            pltpu.make_async_copy(
                rows_v.at[b], out_hbm.at[pl.ds(0, chunk)], osem[b]
            ).wait()

        for b in range(nbuf):
            idx_start(b, b)
        idx_wait(0)
        gather_start(0)

        for b in range(1, nbuf):
            idx_wait(b)
            gather_start(b)
            gather_wait(b - 1)
            out_start(b - 1, b - 1)
            idx_start(b - 1 + nbuf, b - 1)

        def outer(o, carry):
            for b in range(nbuf):
                g = o * nbuf + b
                bp = (b - 1) % nbuf
                idx_wait(b)
                out_wait(b)
                gather_start(b)
                gather_wait(bp)
                out_start(g - 1, bp)
                idx_start(g - 1 + nbuf, bp)
            return carry

        lax.fori_loop(1, n_outer - 1, outer, 0)

        for b in range(nbuf):
            g = (n_outer - 1) * nbuf + b
            bp = (b - 1) % nbuf
            idx_wait(b)
            out_wait(b)
            gather_start(b)
            gather_wait(bp)
            out_start(g - 1, bp)
            if b == 0:
                idx_start(g - 1 + nbuf, bp)

        blast = nbuf - 1
        gather_wait(blast)
        out_start(n_chunks - 1, blast)
        for b in range(nbuf):
            out_wait(b)

    return gather_kernel


def kernel(token_ids, embedding_table):
    n_seq, n_pos = token_ids.shape
    B = n_seq * n_pos
    idx = token_ids.T.reshape(B).astype(jnp.int32)
    out = _make_gather(B, 800, 4)(idx, embedding_table)
    return out.reshape(n_pos, n_seq, DIM).transpose(1, 0, 2)
